# baseline (speedup 1.0000x reference)
import functools

import jax
import jax.numpy as jnp
from jax import lax
from jax.experimental import pallas as pl
from jax.experimental.pallas import tpu as pltpu
from jax.experimental.pallas import tpu_sc as plsc

_N = 10000
_E = 320000
_D = 128
_EPS = 1e-5

_NC = 2
_NS = 16
_NW = _NC * _NS
_NP = 10240
_RPT = _NP // _NS
_CHK = 128
_NG = 2
_GCH = 40
_EPAD = _NW * _NG * _GCH * _CHK

_BLK = 1024
_NBLK = _NP // _BLK

_SQRT_HALF = 0.7071067811865476


def _sc_mesh():
    return plsc.VectorSubcoreMesh(
        core_axis_name="c", subcore_axis_name="s",
        num_cores=_NC, num_subcores=_NS)


def _sc_degree(dstw, ones_hbm, zrows):
    @functools.partial(
        pl.kernel,
        out_type=jax.ShapeDtypeStruct((_NC, _NP, _D), jnp.float32),
        mesh=_sc_mesh(),
        scratch_types=[
            pltpu.VMEM((_NG, _GCH, _CHK), jnp.int32),
            pltpu.VMEM((_CHK, _D), jnp.float32),
            pltpu.VMEM_SHARED((_NP, _D), jnp.float32),
        ],
    )
    def deg_kernel(dstw_ref, ones_ref, zeros_ref, degp_ref, dst_v, ones_v, hist_sh):
        c = lax.axis_index("c")
        s = lax.axis_index("s")
        wid = c * _NS + s
        pltpu.sync_copy(zeros_ref, hist_sh.at[pl.ds(s * _RPT, _RPT)])
        pltpu.sync_copy(dstw_ref.at[wid], dst_v)
        pltpu.sync_copy(ones_ref, ones_v)
        plsc.subcore_barrier()

        for g in range(_NG):
            @pl.loop(0, _GCH)
            def _(k):
                pltpu.sync_copy(ones_v, hist_sh.at[dst_v.at[g, k]], add=True)

        plsc.subcore_barrier()
        pltpu.sync_copy(hist_sh.at[pl.ds(s * _RPT, _RPT)],
                        degp_ref.at[c, pl.ds(s * _RPT, _RPT)])

    return deg_kernel(dstw, ones_hbm, zrows)


def _sc_scatter(srcw, dstw, y, zrows):
    @functools.partial(
        pl.kernel,
        out_type=jax.ShapeDtypeStruct((_NC, _NP, _D), jnp.float32),
        mesh=_sc_mesh(),
        scratch_types=[
            pltpu.VMEM((_GCH, _CHK), jnp.int32),
            pltpu.VMEM((_GCH, _CHK), jnp.int32),
            pltpu.VMEM((_CHK, _D), jnp.float32),
            pltpu.VMEM((_CHK, _D), jnp.float32),
            pltpu.VMEM_SHARED((_NP, _D), jnp.float32),
            pltpu.SemaphoreType.DMA,
            pltpu.SemaphoreType.DMA,
        ],
    )
    def scat_kernel(srcw_ref, dstw_ref, y_ref, zr_ref, out_ref,
                    srcg, dstg, buf_a, buf_b, acc_sh, sem_a, sem_b):
        bufs = (buf_a, buf_b)
        sems = (sem_a, sem_b)
        c = lax.axis_index("c")
        s = lax.axis_index("s")
        wid = c * _NS + s
        pltpu.sync_copy(zr_ref, acc_sh.at[pl.ds(s * _RPT, _RPT)])
        plsc.subcore_barrier()

        for g in range(_NG):
            pltpu.sync_copy(srcw_ref.at[wid, g], srcg)
            pltpu.sync_copy(dstw_ref.at[wid, g], dstg)
            cps = {0: pltpu.async_copy(y_ref.at[srcg.at[0]], bufs[0], sems[0])}
            for k in range(_GCH):
                if k + 1 < _GCH:
                    cps[k + 1] = pltpu.async_copy(
                        y_ref.at[srcg.at[k + 1]], bufs[(k + 1) % 2],
                        sems[(k + 1) % 2])
                cps[k].wait()
                pltpu.sync_copy(bufs[k % 2], acc_sh.at[dstg.at[k]], add=True)

        plsc.subcore_barrier()
        pltpu.sync_copy(acc_sh.at[pl.ds(s * _RPT, _RPT)],
                        out_ref.at[c, pl.ds(s * _RPT, _RPT)])

    return scat_kernel(srcw, dstw, y, zrows)


def _dis_block(degp_blk):
    deg = degp_blk[0] + degp_blk[1] + 1.0
    return lax.rsqrt(deg[:, 0:1])


def _tc_mk_dis(degp):
    def body(degp_ref, dis_ref):
        dis_ref[...] = _dis_block(degp_ref)

    return pl.pallas_call(
        body,
        grid=(_NBLK,),
        in_specs=[pl.BlockSpec((_NC, _BLK, _D), lambda i: (0, i, 0))],
        out_specs=pl.BlockSpec((_BLK, 1), lambda i: (i, 0)),
        out_shape=jax.ShapeDtypeStruct((_NP, 1), jnp.float32),
    )(degp)


def _tc_prep(x, W1, dis):
    def body(x_ref, w_ref, dis_ref, y_ref):
        dis = dis_ref[...]
        y_ref[...] = jnp.dot(x_ref[...], w_ref[...],
                             preferred_element_type=jnp.float32,
                             precision=lax.Precision.HIGHEST) * dis

    return pl.pallas_call(
        body,
        grid=(_NBLK,),
        in_specs=[
            pl.BlockSpec((_BLK, _D), lambda i: (i, 0)),
            pl.BlockSpec((_D, _D), lambda i: (0, 0)),
            pl.BlockSpec((_BLK, 1), lambda i: (i, 0)),
        ],
        out_specs=pl.BlockSpec((_BLK, _D), lambda i: (i, 0)),
        out_shape=jax.ShapeDtypeStruct((_NP, _D), jnp.float32),
    )(x, W1, dis)


def _tc_assemble(Sp, y, dis_v, b):
    def body(sp_ref, y_ref, dis_ref, b_ref, t_ref, st_ref):
        i = pl.program_id(0)
        dis = dis_ref[...]
        t = (sp_ref[0] + sp_ref[1] + y_ref[...]) * dis + b_ref[...]
        t_ref[...] = t

        rid = lax.broadcasted_iota(jnp.int32, (_BLK, 1), 0) + i * _BLK
        tm = jnp.where(rid < _N, t, 0.0)

        @pl.when(i == 0)
        def _():
            st_ref[...] = jnp.zeros_like(st_ref)

        st_ref[0:1, :] += jnp.sum(tm, axis=0, keepdims=True)
        st_ref[1:2, :] += jnp.sum(tm * tm, axis=0, keepdims=True)

    return pl.pallas_call(
        body,
        grid=(_NBLK,),
        in_specs=[
            pl.BlockSpec((_NC, _BLK, _D), lambda i: (0, i, 0)),
            pl.BlockSpec((_BLK, _D), lambda i: (i, 0)),
            pl.BlockSpec((_BLK, 1), lambda i: (i, 0)),
            pl.BlockSpec((1, _D), lambda i: (0, 0)),
        ],
        out_specs=[
            pl.BlockSpec((_BLK, _D), lambda i: (i, 0)),
            pl.BlockSpec((8, _D), lambda i: (0, 0)),
        ],
        out_shape=[
            jax.ShapeDtypeStruct((_NP, _D), jnp.float32),
            jax.ShapeDtypeStruct((8, _D), jnp.float32),
        ],
    )(Sp, y, dis_v, b)


def _tc_bn_gelu_mm(t, st, g, be, W, dis_v):
    def body(t_ref, st_ref, g_ref, be_ref, w_ref, dis_ref, y_ref):
        dis = dis_ref[...]
        mean = st_ref[0:1, :] * (1.0 / _N)
        var = st_ref[1:2, :] * (1.0 / _N) - mean * mean
        h = g_ref[...] * (t_ref[...] - mean) * lax.rsqrt(var + _EPS) + be_ref[...]
        h = 0.5 * h * (1.0 + lax.erf(h * _SQRT_HALF))
        y_ref[...] = jnp.dot(h, w_ref[...],
                             preferred_element_type=jnp.float32,
                             precision=lax.Precision.HIGHEST) * dis

    return pl.pallas_call(
        body,
        grid=(_NBLK,),
        in_specs=[
            pl.BlockSpec((_BLK, _D), lambda i: (i, 0)),
            pl.BlockSpec((8, _D), lambda i: (0, 0)),
            pl.BlockSpec((1, _D), lambda i: (0, 0)),
            pl.BlockSpec((1, _D), lambda i: (0, 0)),
            pl.BlockSpec((_D, _D), lambda i: (0, 0)),
            pl.BlockSpec((_BLK, 1), lambda i: (i, 0)),
        ],
        out_specs=pl.BlockSpec((_BLK, _D), lambda i: (i, 0)),
        out_shape=jax.ShapeDtypeStruct((_NP, _D), jnp.float32),
    )(t, st, g, be, W, dis_v)


def _tc_final(Sp, y, dis_v, b):
    def body(sp_ref, y_ref, dis_ref, b_ref, o_ref):
        dis = dis_ref[...]
        o_ref[...] = (sp_ref[0] + sp_ref[1] + y_ref[...]) * dis + b_ref[...]

    return pl.pallas_call(
        body,
        grid=(_NBLK,),
        in_specs=[
            pl.BlockSpec((_NC, _BLK, _D), lambda i: (0, i, 0)),
            pl.BlockSpec((_BLK, _D), lambda i: (i, 0)),
            pl.BlockSpec((_BLK, 1), lambda i: (i, 0)),
            pl.BlockSpec((1, _D), lambda i: (0, 0)),
        ],
        out_specs=pl.BlockSpec((_BLK, _D), lambda i: (i, 0)),
        out_shape=jax.ShapeDtypeStruct((_NP, _D), jnp.float32),
    )(Sp, y, dis_v, b)


def kernel(x, edge_index, W1, b1, g1, be1, W2, b2, g2, be2, W3, b3):
    npad = _EPAD - _E
    pad_idx = _N + (jnp.arange(npad, dtype=jnp.int32) % (_NP - _N))
    src = jnp.concatenate([edge_index[0], pad_idx]).reshape(_NW, _NG, _GCH, _CHK)
    dst = jnp.concatenate([edge_index[1], pad_idx]).reshape(_NW, _NG, _GCH, _CHK)
    xp = jnp.pad(x, ((0, _NP - _N), (0, 0)))
    ones_hbm = jnp.ones((_CHK, _D), jnp.float32)
    zrows = jnp.zeros((_RPT, _D), jnp.float32)

    degp = _sc_degree(dst, ones_hbm, zrows)
    dis_v = _tc_mk_dis(degp)

    y1 = _tc_prep(xp, W1, dis_v)
    S1 = _sc_scatter(src, dst, y1, zrows)
    t1, st1 = _tc_assemble(S1, y1, dis_v, b1.reshape(1, _D))
    y2 = _tc_bn_gelu_mm(t1, st1, g1.reshape(1, _D), be1.reshape(1, _D), W2, dis_v)

    S2 = _sc_scatter(src, dst, y2, zrows)
    t2, st2 = _tc_assemble(S2, y2, dis_v, b2.reshape(1, _D))
    y3 = _tc_bn_gelu_mm(t2, st2, g2.reshape(1, _D), be2.reshape(1, _D), W3, dis_v)

    S3 = _sc_scatter(src, dst, y3, zrows)
    out = _tc_final(S3, y3, dis_v, b3.reshape(1, _D))
    return out[:_N]


# speedup vs baseline: 1.0163x; 1.0163x over previous
import functools

import jax
import jax.numpy as jnp
from jax import lax
from jax.experimental import pallas as pl
from jax.experimental.pallas import tpu as pltpu
from jax.experimental.pallas import tpu_sc as plsc

_N = 10000
_E = 320000
_D = 128
_EPS = 1e-5

_NC = 2
_NS = 16
_NW = _NC * _NS
_NP = 10240
_RPT = _NP // _NS
_CHK = 128
_NG = 2
_GCH = 40
_EPAD = _NW * _NG * _GCH * _CHK

_BLK = 1024
_NBLK = _NP // _BLK

_SQRT_HALF = 0.7071067811865476


def _sc_mesh():
    return plsc.VectorSubcoreMesh(
        core_axis_name="c", subcore_axis_name="s",
        num_cores=_NC, num_subcores=_NS)


def _sc_degree(dstw, ones_hbm, zrows):
    @functools.partial(
        pl.kernel,
        out_type=jax.ShapeDtypeStruct((_NC, _NP, _D), jnp.float32),
        mesh=_sc_mesh(),
        scratch_types=[
            pltpu.VMEM((_NG, _GCH, _CHK), jnp.int32),
            pltpu.VMEM((_CHK, _D), jnp.float32),
            pltpu.VMEM_SHARED((_NP, _D), jnp.float32),
        ],
    )
    def deg_kernel(dstw_ref, ones_ref, zeros_ref, degp_ref, dst_v, ones_v, hist_sh):
        c = lax.axis_index("c")
        s = lax.axis_index("s")
        wid = c * _NS + s
        pltpu.sync_copy(zeros_ref, hist_sh.at[pl.ds(s * _RPT, _RPT)])
        pltpu.sync_copy(dstw_ref.at[wid], dst_v)
        pltpu.sync_copy(ones_ref, ones_v)
        plsc.subcore_barrier()

        for g in range(_NG):
            @pl.loop(0, _GCH)
            def _(k):
                pltpu.sync_copy(ones_v, hist_sh.at[dst_v.at[g, k]], add=True)

        plsc.subcore_barrier()
        pltpu.sync_copy(hist_sh.at[pl.ds(s * _RPT, _RPT)],
                        degp_ref.at[c, pl.ds(s * _RPT, _RPT)])

    return deg_kernel(dstw, ones_hbm, zrows)


def _sc_scatter(srcw, dstw, y, zrows):
    @functools.partial(
        pl.kernel,
        out_type=jax.ShapeDtypeStruct((_NC, _NP, _D), jnp.float32),
        mesh=_sc_mesh(),
        scratch_types=[
            pltpu.VMEM((_GCH, _CHK), jnp.int32),
            pltpu.VMEM((_GCH, _CHK), jnp.int32),
            pltpu.VMEM((_CHK, _D), jnp.float32),
            pltpu.VMEM((_CHK, _D), jnp.float32),
            pltpu.VMEM_SHARED((_NP, _D), jnp.float32),
            pltpu.SemaphoreType.DMA,
            pltpu.SemaphoreType.DMA,
            pltpu.SemaphoreType.DMA,
        ],
    )
    def scat_kernel(srcw_ref, dstw_ref, y_ref, zr_ref, out_ref,
                    srcg, dstg, buf_a, buf_b, acc_sh, sem_a, sem_b, sem_z):
        bufs = (buf_a, buf_b)
        sems = (sem_a, sem_b)
        c = lax.axis_index("c")
        s = lax.axis_index("s")
        wid = c * _NS + s
        zcp = pltpu.async_copy(zr_ref, acc_sh.at[pl.ds(s * _RPT, _RPT)], sem_z)

        for g in range(_NG):
            pltpu.sync_copy(srcw_ref.at[wid, g], srcg)
            pltpu.sync_copy(dstw_ref.at[wid, g], dstg)
            cps = {0: pltpu.async_copy(y_ref.at[srcg.at[0]], bufs[0], sems[0])}
            if g == 0:
                zcp.wait()
                plsc.subcore_barrier()
            for k in range(_GCH):
                if k + 1 < _GCH:
                    cps[k + 1] = pltpu.async_copy(
                        y_ref.at[srcg.at[k + 1]], bufs[(k + 1) % 2],
                        sems[(k + 1) % 2])
                cps[k].wait()
                pltpu.sync_copy(bufs[k % 2], acc_sh.at[dstg.at[k]], add=True)

        plsc.subcore_barrier()
        pltpu.sync_copy(acc_sh.at[pl.ds(s * _RPT, _RPT)],
                        out_ref.at[c, pl.ds(s * _RPT, _RPT)])

    return scat_kernel(srcw, dstw, y, zrows)


def _dis_block(degp_blk):
    deg = degp_blk[0] + degp_blk[1] + 1.0
    return lax.rsqrt(deg[:, 0:1])


def _tc_mk_dis(degp):
    def body(degp_ref, dis_ref):
        dis_ref[...] = _dis_block(degp_ref)

    return pl.pallas_call(
        body,
        grid=(_NBLK,),
        in_specs=[pl.BlockSpec((_NC, _BLK, _D), lambda i: (0, i, 0))],
        out_specs=pl.BlockSpec((_BLK, 1), lambda i: (i, 0)),
        out_shape=jax.ShapeDtypeStruct((_NP, 1), jnp.float32),
    )(degp)


def _tc_prep(x, W1, dis):
    def body(x_ref, w_ref, dis_ref, y_ref):
        dis = dis_ref[...]
        y_ref[...] = jnp.dot(x_ref[...], w_ref[...],
                             preferred_element_type=jnp.float32,
                             precision=lax.Precision.HIGHEST) * dis

    return pl.pallas_call(
        body,
        grid=(_NBLK,),
        in_specs=[
            pl.BlockSpec((_BLK, _D), lambda i: (i, 0)),
            pl.BlockSpec((_D, _D), lambda i: (0, 0)),
            pl.BlockSpec((_BLK, 1), lambda i: (i, 0)),
        ],
        out_specs=pl.BlockSpec((_BLK, _D), lambda i: (i, 0)),
        out_shape=jax.ShapeDtypeStruct((_NP, _D), jnp.float32),
    )(x, W1, dis)


def _tc_assemble(Sp, y, dis_v, b):
    def body(sp_ref, y_ref, dis_ref, b_ref, t_ref, st_ref):
        i = pl.program_id(0)
        dis = dis_ref[...]
        t = (sp_ref[0] + sp_ref[1] + y_ref[...]) * dis + b_ref[...]
        t_ref[...] = t

        rid = lax.broadcasted_iota(jnp.int32, (_BLK, 1), 0) + i * _BLK
        tm = jnp.where(rid < _N, t, 0.0)

        @pl.when(i == 0)
        def _():
            st_ref[...] = jnp.zeros_like(st_ref)

        st_ref[0:1, :] += jnp.sum(tm, axis=0, keepdims=True)
        st_ref[1:2, :] += jnp.sum(tm * tm, axis=0, keepdims=True)

    return pl.pallas_call(
        body,
        grid=(_NBLK,),
        in_specs=[
            pl.BlockSpec((_NC, _BLK, _D), lambda i: (0, i, 0)),
            pl.BlockSpec((_BLK, _D), lambda i: (i, 0)),
            pl.BlockSpec((_BLK, 1), lambda i: (i, 0)),
            pl.BlockSpec((1, _D), lambda i: (0, 0)),
        ],
        out_specs=[
            pl.BlockSpec((_BLK, _D), lambda i: (i, 0)),
            pl.BlockSpec((8, _D), lambda i: (0, 0)),
        ],
        out_shape=[
            jax.ShapeDtypeStruct((_NP, _D), jnp.float32),
            jax.ShapeDtypeStruct((8, _D), jnp.float32),
        ],
    )(Sp, y, dis_v, b)


def _tc_bn_gelu_mm(t, st, g, be, W, dis_v):
    def body(t_ref, st_ref, g_ref, be_ref, w_ref, dis_ref, y_ref):
        dis = dis_ref[...]
        mean = st_ref[0:1, :] * (1.0 / _N)
        var = st_ref[1:2, :] * (1.0 / _N) - mean * mean
        h = g_ref[...] * (t_ref[...] - mean) * lax.rsqrt(var + _EPS) + be_ref[...]
        h = 0.5 * h * (1.0 + lax.erf(h * _SQRT_HALF))
        y_ref[...] = jnp.dot(h, w_ref[...],
                             preferred_element_type=jnp.float32,
                             precision=lax.Precision.HIGHEST) * dis

    return pl.pallas_call(
        body,
        grid=(_NBLK,),
        in_specs=[
            pl.BlockSpec((_BLK, _D), lambda i: (i, 0)),
            pl.BlockSpec((8, _D), lambda i: (0, 0)),
            pl.BlockSpec((1, _D), lambda i: (0, 0)),
            pl.BlockSpec((1, _D), lambda i: (0, 0)),
            pl.BlockSpec((_D, _D), lambda i: (0, 0)),
            pl.BlockSpec((_BLK, 1), lambda i: (i, 0)),
        ],
        out_specs=pl.BlockSpec((_BLK, _D), lambda i: (i, 0)),
        out_shape=jax.ShapeDtypeStruct((_NP, _D), jnp.float32),
    )(t, st, g, be, W, dis_v)


def _tc_final(Sp, y, dis_v, b):
    def body(sp_ref, y_ref, dis_ref, b_ref, o_ref):
        dis = dis_ref[...]
        o_ref[...] = (sp_ref[0] + sp_ref[1] + y_ref[...]) * dis + b_ref[...]

    return pl.pallas_call(
        body,
        grid=(_NBLK,),
        in_specs=[
            pl.BlockSpec((_NC, _BLK, _D), lambda i: (0, i, 0)),
            pl.BlockSpec((_BLK, _D), lambda i: (i, 0)),
            pl.BlockSpec((_BLK, 1), lambda i: (i, 0)),
            pl.BlockSpec((1, _D), lambda i: (0, 0)),
        ],
        out_specs=pl.BlockSpec((_BLK, _D), lambda i: (i, 0)),
        out_shape=jax.ShapeDtypeStruct((_NP, _D), jnp.float32),
    )(Sp, y, dis_v, b)


def kernel(x, edge_index, W1, b1, g1, be1, W2, b2, g2, be2, W3, b3):
    npad = _EPAD - _E
    pad_idx = _N + (jnp.arange(npad, dtype=jnp.int32) % (_NP - _N))
    src = jnp.concatenate([edge_index[0], pad_idx]).reshape(_NW, _NG, _GCH, _CHK)
    dst = jnp.concatenate([edge_index[1], pad_idx]).reshape(_NW, _NG, _GCH, _CHK)
    xp = jnp.pad(x, ((0, _NP - _N), (0, 0)))
    ones_hbm = jnp.ones((_CHK, _D), jnp.float32)
    zrows = jnp.zeros((_RPT, _D), jnp.float32)

    degp = _sc_degree(dst, ones_hbm, zrows)
    dis_v = _tc_mk_dis(degp)

    y1 = _tc_prep(xp, W1, dis_v)
    S1 = _sc_scatter(src, dst, y1, zrows)
    t1, st1 = _tc_assemble(S1, y1, dis_v, b1.reshape(1, _D))
    y2 = _tc_bn_gelu_mm(t1, st1, g1.reshape(1, _D), be1.reshape(1, _D), W2, dis_v)

    S2 = _sc_scatter(src, dst, y2, zrows)
    t2, st2 = _tc_assemble(S2, y2, dis_v, b2.reshape(1, _D))
    y3 = _tc_bn_gelu_mm(t2, st2, g2.reshape(1, _D), be2.reshape(1, _D), W3, dis_v)

    S3 = _sc_scatter(src, dst, y3, zrows)
    out = _tc_final(S3, y3, dis_v, b3.reshape(1, _D))
    return out[:_N]


# speedup vs baseline: 1.0499x; 1.0331x over previous
import functools

import jax
import jax.numpy as jnp
from jax import lax
from jax.experimental import pallas as pl
from jax.experimental.pallas import tpu as pltpu
from jax.experimental.pallas import tpu_sc as plsc

_N = 10000
_E = 320000
_D = 128
_EPS = 1e-5

_NC = 2
_NS = 16
_NW = _NC * _NS
_NP = 10240
_RPT = _NP // _NS
_CHK = 128
_NG = 2
_GCH = 40
_EPAD = _NW * _NG * _GCH * _CHK

_BLK = 1024
_NBLK = _NP // _BLK

_SQRT_HALF = 0.7071067811865476


def _sc_mesh():
    return plsc.VectorSubcoreMesh(
        core_axis_name="c", subcore_axis_name="s",
        num_cores=_NC, num_subcores=_NS)


def _sc_degree(dstw, ones_hbm, zrows):
    @functools.partial(
        pl.kernel,
        out_type=jax.ShapeDtypeStruct((_NC, _NP, _D), jnp.float32),
        mesh=_sc_mesh(),
        scratch_types=[
            pltpu.VMEM((_NG, _GCH, _CHK), jnp.int32),
            pltpu.VMEM((_CHK, _D), jnp.float32),
            pltpu.VMEM_SHARED((_NP, _D), jnp.float32),
        ],
    )
    def deg_kernel(dstw_ref, ones_ref, zeros_ref, degp_ref, dst_v, ones_v, hist_sh):
        c = lax.axis_index("c")
        s = lax.axis_index("s")
        wid = c * _NS + s
        pltpu.sync_copy(zeros_ref, hist_sh.at[pl.ds(s * _RPT, _RPT)])
        pltpu.sync_copy(dstw_ref.at[wid], dst_v)
        pltpu.sync_copy(ones_ref, ones_v)
        plsc.subcore_barrier()

        for g in range(_NG):
            @pl.loop(0, _GCH)
            def _(k):
                pltpu.sync_copy(ones_v, hist_sh.at[dst_v.at[g, k]], add=True)

        plsc.subcore_barrier()
        pltpu.sync_copy(hist_sh.at[pl.ds(s * _RPT, _RPT)],
                        degp_ref.at[c, pl.ds(s * _RPT, _RPT)])

    return deg_kernel(dstw, ones_hbm, zrows)


def _sc_scatter(srcw, dstw, y, zrows):
    @functools.partial(
        pl.kernel,
        out_type=jax.ShapeDtypeStruct((_NC, _NP, _D), jnp.float32),
        mesh=_sc_mesh(),
        scratch_types=[
            pltpu.VMEM((_GCH, _CHK), jnp.int32),
            pltpu.VMEM((_GCH, _CHK), jnp.int32),
            pltpu.VMEM((_CHK, _D), jnp.float32),
            pltpu.VMEM((_CHK, _D), jnp.float32),
            pltpu.VMEM_SHARED((_NP, _D), jnp.float32),
            pltpu.SemaphoreType.DMA,
            pltpu.SemaphoreType.DMA,
            pltpu.SemaphoreType.DMA,
        ],
    )
    def scat_kernel(srcw_ref, dstw_ref, y_ref, zr_ref, out_ref,
                    srcg, dstg, buf_a, buf_b, acc_sh, sem_a, sem_b, sem_z):
        bufs = (buf_a, buf_b)
        sems = (sem_a, sem_b)
        c = lax.axis_index("c")
        s = lax.axis_index("s")
        wid = c * _NS + s
        zcp = pltpu.async_copy(zr_ref, acc_sh.at[pl.ds(s * _RPT, _RPT)], sem_z)

        for g in range(_NG):
            pltpu.sync_copy(srcw_ref.at[wid, g], srcg)
            pltpu.sync_copy(dstw_ref.at[wid, g], dstg)
            cps = {0: pltpu.async_copy(y_ref.at[srcg.at[0]], bufs[0], sems[0])}
            if g == 0:
                zcp.wait()
                plsc.subcore_barrier()
            for k in range(_GCH):
                if k + 1 < _GCH:
                    cps[k + 1] = pltpu.async_copy(
                        y_ref.at[srcg.at[k + 1]], bufs[(k + 1) % 2],
                        sems[(k + 1) % 2])
                cps[k].wait()
                pltpu.sync_copy(bufs[k % 2], acc_sh.at[dstg.at[k]], add=True)

        plsc.subcore_barrier()
        pltpu.sync_copy(acc_sh.at[pl.ds(s * _RPT, _RPT)],
                        out_ref.at[c, pl.ds(s * _RPT, _RPT)])

    return scat_kernel(srcw, dstw, y, zrows)


def _dis_block(degp_blk):
    deg = degp_blk[0] + degp_blk[1] + 1.0
    return lax.rsqrt(deg[:, 0:1])


def _tc_prep(x, W1, degp):
    def body(x_ref, w_ref, degp_ref, y_ref, dis_ref):
        dis = _dis_block(degp_ref)
        dis_ref[...] = dis
        y_ref[...] = jnp.dot(x_ref[...], w_ref[...],
                             preferred_element_type=jnp.float32) * dis

    return pl.pallas_call(
        body,
        grid=(_NBLK,),
        in_specs=[
            pl.BlockSpec((_BLK, _D), lambda i: (i, 0)),
            pl.BlockSpec((_D, _D), lambda i: (0, 0)),
            pl.BlockSpec((_NC, _BLK, _D), lambda i: (0, i, 0)),
        ],
        out_specs=[
            pl.BlockSpec((_BLK, _D), lambda i: (i, 0)),
            pl.BlockSpec((_BLK, 1), lambda i: (i, 0)),
        ],
        out_shape=[
            jax.ShapeDtypeStruct((_NP, _D), jnp.float32),
            jax.ShapeDtypeStruct((_NP, 1), jnp.float32),
        ],
    )(x, W1, degp)


def _tc_assemble(Sp, y, dis_v, b):
    def body(sp_ref, y_ref, dis_ref, b_ref, t_ref, st_ref):
        i = pl.program_id(0)
        dis = dis_ref[...]
        t = (sp_ref[0] + sp_ref[1] + y_ref[...]) * dis + b_ref[...]
        t_ref[...] = t

        rid = lax.broadcasted_iota(jnp.int32, (_BLK, 1), 0) + i * _BLK
        tm = jnp.where(rid < _N, t, 0.0)

        @pl.when(i == 0)
        def _():
            st_ref[...] = jnp.zeros_like(st_ref)

        st_ref[0:1, :] += jnp.sum(tm, axis=0, keepdims=True)
        st_ref[1:2, :] += jnp.sum(tm * tm, axis=0, keepdims=True)

    return pl.pallas_call(
        body,
        grid=(_NBLK,),
        in_specs=[
            pl.BlockSpec((_NC, _BLK, _D), lambda i: (0, i, 0)),
            pl.BlockSpec((_BLK, _D), lambda i: (i, 0)),
            pl.BlockSpec((_BLK, 1), lambda i: (i, 0)),
            pl.BlockSpec((1, _D), lambda i: (0, 0)),
        ],
        out_specs=[
            pl.BlockSpec((_BLK, _D), lambda i: (i, 0)),
            pl.BlockSpec((8, _D), lambda i: (0, 0)),
        ],
        out_shape=[
            jax.ShapeDtypeStruct((_NP, _D), jnp.float32),
            jax.ShapeDtypeStruct((8, _D), jnp.float32),
        ],
    )(Sp, y, dis_v, b)


def _tc_bn_gelu_mm(t, st, g, be, W, dis_v):
    def body(t_ref, st_ref, g_ref, be_ref, w_ref, dis_ref, y_ref):
        dis = dis_ref[...]
        mean = st_ref[0:1, :] * (1.0 / _N)
        var = st_ref[1:2, :] * (1.0 / _N) - mean * mean
        h = g_ref[...] * (t_ref[...] - mean) * lax.rsqrt(var + _EPS) + be_ref[...]
        h = 0.5 * h * (1.0 + lax.erf(h * _SQRT_HALF))
        y_ref[...] = jnp.dot(h, w_ref[...],
                             preferred_element_type=jnp.float32) * dis

    return pl.pallas_call(
        body,
        grid=(_NBLK,),
        in_specs=[
            pl.BlockSpec((_BLK, _D), lambda i: (i, 0)),
            pl.BlockSpec((8, _D), lambda i: (0, 0)),
            pl.BlockSpec((1, _D), lambda i: (0, 0)),
            pl.BlockSpec((1, _D), lambda i: (0, 0)),
            pl.BlockSpec((_D, _D), lambda i: (0, 0)),
            pl.BlockSpec((_BLK, 1), lambda i: (i, 0)),
        ],
        out_specs=pl.BlockSpec((_BLK, _D), lambda i: (i, 0)),
        out_shape=jax.ShapeDtypeStruct((_NP, _D), jnp.float32),
    )(t, st, g, be, W, dis_v)


def _tc_final(Sp, y, dis_v, b):
    def body(sp_ref, y_ref, dis_ref, b_ref, o_ref):
        dis = dis_ref[...]
        o_ref[...] = (sp_ref[0] + sp_ref[1] + y_ref[...]) * dis + b_ref[...]

    return pl.pallas_call(
        body,
        grid=(_NBLK,),
        in_specs=[
            pl.BlockSpec((_NC, _BLK, _D), lambda i: (0, i, 0)),
            pl.BlockSpec((_BLK, _D), lambda i: (i, 0)),
            pl.BlockSpec((_BLK, 1), lambda i: (i, 0)),
            pl.BlockSpec((1, _D), lambda i: (0, 0)),
        ],
        out_specs=pl.BlockSpec((_BLK, _D), lambda i: (i, 0)),
        out_shape=jax.ShapeDtypeStruct((_N, _D), jnp.float32),
    )(Sp, y, dis_v, b)


def kernel(x, edge_index, W1, b1, g1, be1, W2, b2, g2, be2, W3, b3):
    npad = _EPAD - _E
    pad_idx = _N + (jnp.arange(npad, dtype=jnp.int32) % (_NP - _N))
    src = jnp.concatenate([edge_index[0], pad_idx]).reshape(_NW, _NG, _GCH, _CHK)
    dst = jnp.concatenate([edge_index[1], pad_idx]).reshape(_NW, _NG, _GCH, _CHK)
    ones_hbm = jnp.ones((_CHK, _D), jnp.float32)
    zrows = jnp.zeros((_RPT, _D), jnp.float32)

    degp = _sc_degree(dst, ones_hbm, zrows)

    y1, dis_v = _tc_prep(x, W1, degp)
    S1 = _sc_scatter(src, dst, y1, zrows)
    t1, st1 = _tc_assemble(S1, y1, dis_v, b1.reshape(1, _D))
    y2 = _tc_bn_gelu_mm(t1, st1, g1.reshape(1, _D), be1.reshape(1, _D), W2, dis_v)

    S2 = _sc_scatter(src, dst, y2, zrows)
    t2, st2 = _tc_assemble(S2, y2, dis_v, b2.reshape(1, _D))
    y3 = _tc_bn_gelu_mm(t2, st2, g2.reshape(1, _D), be2.reshape(1, _D), W3, dis_v)

    S3 = _sc_scatter(src, dst, y3, zrows)
    return _tc_final(S3, y3, dis_v, b3.reshape(1, _D))


# speedup vs baseline: 1.0546x; 1.0045x over previous
import functools

import jax
import jax.numpy as jnp
from jax import lax
from jax.experimental import pallas as pl
from jax.experimental.pallas import tpu as pltpu
from jax.experimental.pallas import tpu_sc as plsc

_N = 10000
_E = 320000
_D = 128
_EPS = 1e-5

_NC = 2
_NS = 16
_NW = _NC * _NS
_NP = 10240
_RPT = _NP // _NS
_CHK = 128
_NG = 2
_GCH = 40
_EPAD = _NW * _NG * _GCH * _CHK

_BLK = 1024
_NBLK = _NP // _BLK

_SQRT_HALF = 0.7071067811865476


def _sc_mesh():
    return plsc.VectorSubcoreMesh(
        core_axis_name="c", subcore_axis_name="s",
        num_cores=_NC, num_subcores=_NS)


def _sc_degree(dstw, ones_hbm, zrows):
    @functools.partial(
        pl.kernel,
        out_type=jax.ShapeDtypeStruct((_NC, _NP, _D), jnp.float32),
        mesh=_sc_mesh(),
        scratch_types=[
            pltpu.VMEM((_NG, _GCH, _CHK), jnp.int32),
            pltpu.VMEM((_CHK, _D), jnp.float32),
            pltpu.VMEM_SHARED((_NP, _D), jnp.float32),
            pltpu.SemaphoreType.DMA,
        ],
    )
    def deg_kernel(dstw_ref, ones_ref, zeros_ref, degp_ref, dst_v, ones_v, hist_sh,
                   sem):
        c = lax.axis_index("c")
        s = lax.axis_index("s")
        wid = c * _NS + s
        pltpu.sync_copy(zeros_ref, hist_sh.at[pl.ds(s * _RPT, _RPT)])
        pltpu.sync_copy(dstw_ref.at[wid], dst_v)
        pltpu.sync_copy(ones_ref, ones_v)
        plsc.subcore_barrier()

        cps = [pltpu.async_copy(ones_v, hist_sh.at[dst_v.at[g, k]], sem, add=True)
               for g in range(_NG) for k in range(_GCH)]
        for cp in cps:
            cp.wait()

        plsc.subcore_barrier()
        pltpu.sync_copy(hist_sh.at[pl.ds(s * _RPT, _RPT)],
                        degp_ref.at[c, pl.ds(s * _RPT, _RPT)])

    return deg_kernel(dstw, ones_hbm, zrows)


def _sc_scatter(srcw, dstw, y, zrows):
    @functools.partial(
        pl.kernel,
        out_type=jax.ShapeDtypeStruct((_NC, _NP, _D), jnp.float32),
        mesh=_sc_mesh(),
        scratch_types=[
            pltpu.VMEM((_GCH, _CHK), jnp.int32),
            pltpu.VMEM((_GCH, _CHK), jnp.int32),
            pltpu.VMEM((_CHK, _D), jnp.float32),
            pltpu.VMEM((_CHK, _D), jnp.float32),
            pltpu.VMEM_SHARED((_NP, _D), jnp.float32),
            pltpu.SemaphoreType.DMA,
            pltpu.SemaphoreType.DMA,
            pltpu.SemaphoreType.DMA,
            pltpu.SemaphoreType.DMA,
            pltpu.SemaphoreType.DMA,
        ],
    )
    def scat_kernel(srcw_ref, dstw_ref, y_ref, zr_ref, out_ref,
                    srcg, dstg, buf_a, buf_b, acc_sh,
                    sem_a, sem_b, ssem_a, ssem_b, sem_z):
        bufs = (buf_a, buf_b)
        sems = (sem_a, sem_b)
        ssems = (ssem_a, ssem_b)
        c = lax.axis_index("c")
        s = lax.axis_index("s")
        wid = c * _NS + s
        zcp = pltpu.async_copy(zr_ref, acc_sh.at[pl.ds(s * _RPT, _RPT)], sem_z)

        for g in range(_NG):
            pltpu.sync_copy(srcw_ref.at[wid, g], srcg)
            pltpu.sync_copy(dstw_ref.at[wid, g], dstg)
            cps = {0: pltpu.async_copy(y_ref.at[srcg.at[0]], bufs[0], sems[0])}
            if g == 0:
                zcp.wait()
                plsc.subcore_barrier()
            scps = {}
            for k in range(_GCH):
                if k + 1 < _GCH:
                    if k >= 1:
                        scps[k - 1].wait()
                    cps[k + 1] = pltpu.async_copy(
                        y_ref.at[srcg.at[k + 1]], bufs[(k + 1) % 2],
                        sems[(k + 1) % 2])
                cps[k].wait()
                scps[k] = pltpu.async_copy(
                    bufs[k % 2], acc_sh.at[dstg.at[k]], ssems[k % 2], add=True)
            scps[_GCH - 2].wait()
            scps[_GCH - 1].wait()

        plsc.subcore_barrier()
        pltpu.sync_copy(acc_sh.at[pl.ds(s * _RPT, _RPT)],
                        out_ref.at[c, pl.ds(s * _RPT, _RPT)])

    return scat_kernel(srcw, dstw, y, zrows)


def _dis_block(degp_blk):
    deg = degp_blk[0] + degp_blk[1] + 1.0
    return lax.rsqrt(deg[:, 0:1])


def _tc_prep(x, W1, degp):
    def body(x_ref, w_ref, degp_ref, y_ref, dis_ref):
        dis = _dis_block(degp_ref)
        dis_ref[...] = dis
        y_ref[...] = jnp.dot(x_ref[...], w_ref[...],
                             preferred_element_type=jnp.float32) * dis

    return pl.pallas_call(
        body,
        grid=(_NBLK,),
        in_specs=[
            pl.BlockSpec((_BLK, _D), lambda i: (i, 0)),
            pl.BlockSpec((_D, _D), lambda i: (0, 0)),
            pl.BlockSpec((_NC, _BLK, _D), lambda i: (0, i, 0)),
        ],
        out_specs=[
            pl.BlockSpec((_BLK, _D), lambda i: (i, 0)),
            pl.BlockSpec((_BLK, 1), lambda i: (i, 0)),
        ],
        out_shape=[
            jax.ShapeDtypeStruct((_NP, _D), jnp.float32),
            jax.ShapeDtypeStruct((_NP, 1), jnp.float32),
        ],
    )(x, W1, degp)


def _tc_assemble(Sp, y, dis_v, b):
    def body(sp_ref, y_ref, dis_ref, b_ref, t_ref, st_ref):
        i = pl.program_id(0)
        dis = dis_ref[...]
        t = (sp_ref[0] + sp_ref[1] + y_ref[...]) * dis + b_ref[...]
        t_ref[...] = t

        rid = lax.broadcasted_iota(jnp.int32, (_BLK, 1), 0) + i * _BLK
        tm = jnp.where(rid < _N, t, 0.0)

        @pl.when(i == 0)
        def _():
            st_ref[...] = jnp.zeros_like(st_ref)

        st_ref[0:1, :] += jnp.sum(tm, axis=0, keepdims=True)
        st_ref[1:2, :] += jnp.sum(tm * tm, axis=0, keepdims=True)

    return pl.pallas_call(
        body,
        grid=(_NBLK,),
        in_specs=[
            pl.BlockSpec((_NC, _BLK, _D), lambda i: (0, i, 0)),
            pl.BlockSpec((_BLK, _D), lambda i: (i, 0)),
            pl.BlockSpec((_BLK, 1), lambda i: (i, 0)),
            pl.BlockSpec((1, _D), lambda i: (0, 0)),
        ],
        out_specs=[
            pl.BlockSpec((_BLK, _D), lambda i: (i, 0)),
            pl.BlockSpec((8, _D), lambda i: (0, 0)),
        ],
        out_shape=[
            jax.ShapeDtypeStruct((_NP, _D), jnp.float32),
            jax.ShapeDtypeStruct((8, _D), jnp.float32),
        ],
    )(Sp, y, dis_v, b)


def _tc_bn_gelu_mm(t, st, g, be, W, dis_v):
    def body(t_ref, st_ref, g_ref, be_ref, w_ref, dis_ref, y_ref):
        dis = dis_ref[...]
        mean = st_ref[0:1, :] * (1.0 / _N)
        var = st_ref[1:2, :] * (1.0 / _N) - mean * mean
        h = g_ref[...] * (t_ref[...] - mean) * lax.rsqrt(var + _EPS) + be_ref[...]
        h = 0.5 * h * (1.0 + lax.erf(h * _SQRT_HALF))
        y_ref[...] = jnp.dot(h, w_ref[...],
                             preferred_element_type=jnp.float32) * dis

    return pl.pallas_call(
        body,
        grid=(_NBLK,),
        in_specs=[
            pl.BlockSpec((_BLK, _D), lambda i: (i, 0)),
            pl.BlockSpec((8, _D), lambda i: (0, 0)),
            pl.BlockSpec((1, _D), lambda i: (0, 0)),
            pl.BlockSpec((1, _D), lambda i: (0, 0)),
            pl.BlockSpec((_D, _D), lambda i: (0, 0)),
            pl.BlockSpec((_BLK, 1), lambda i: (i, 0)),
        ],
        out_specs=pl.BlockSpec((_BLK, _D), lambda i: (i, 0)),
        out_shape=jax.ShapeDtypeStruct((_NP, _D), jnp.float32),
    )(t, st, g, be, W, dis_v)


def _tc_final(Sp, y, dis_v, b):
    def body(sp_ref, y_ref, dis_ref, b_ref, o_ref):
        dis = dis_ref[...]
        o_ref[...] = (sp_ref[0] + sp_ref[1] + y_ref[...]) * dis + b_ref[...]

    return pl.pallas_call(
        body,
        grid=(_NBLK,),
        in_specs=[
            pl.BlockSpec((_NC, _BLK, _D), lambda i: (0, i, 0)),
            pl.BlockSpec((_BLK, _D), lambda i: (i, 0)),
            pl.BlockSpec((_BLK, 1), lambda i: (i, 0)),
            pl.BlockSpec((1, _D), lambda i: (0, 0)),
        ],
        out_specs=pl.BlockSpec((_BLK, _D), lambda i: (i, 0)),
        out_shape=jax.ShapeDtypeStruct((_N, _D), jnp.float32),
    )(Sp, y, dis_v, b)


def kernel(x, edge_index, W1, b1, g1, be1, W2, b2, g2, be2, W3, b3):
    npad = _EPAD - _E
    pad_idx = _N + (jnp.arange(npad, dtype=jnp.int32) % (_NP - _N))
    src = jnp.concatenate([edge_index[0], pad_idx]).reshape(_NW, _NG, _GCH, _CHK)
    dst = jnp.concatenate([edge_index[1], pad_idx]).reshape(_NW, _NG, _GCH, _CHK)
    ones_hbm = jnp.ones((_CHK, _D), jnp.float32)
    zrows = jnp.zeros((_RPT, _D), jnp.float32)

    degp = _sc_degree(dst, ones_hbm, zrows)

    y1, dis_v = _tc_prep(x, W1, degp)
    S1 = _sc_scatter(src, dst, y1, zrows)
    t1, st1 = _tc_assemble(S1, y1, dis_v, b1.reshape(1, _D))
    y2 = _tc_bn_gelu_mm(t1, st1, g1.reshape(1, _D), be1.reshape(1, _D), W2, dis_v)

    S2 = _sc_scatter(src, dst, y2, zrows)
    t2, st2 = _tc_assemble(S2, y2, dis_v, b2.reshape(1, _D))
    y3 = _tc_bn_gelu_mm(t2, st2, g2.reshape(1, _D), be2.reshape(1, _D), W3, dis_v)

    S3 = _sc_scatter(src, dst, y3, zrows)
    return _tc_final(S3, y3, dis_v, b3.reshape(1, _D))


# speedup vs baseline: 1.0912x; 1.0347x over previous
import functools

import jax
import jax.numpy as jnp
from jax import lax
from jax.experimental import pallas as pl
from jax.experimental.pallas import tpu as pltpu
from jax.experimental.pallas import tpu_sc as plsc

_N = 10000
_E = 320000
_D = 128
_EPS = 1e-5

_NC = 2
_NS = 16
_NW = _NC * _NS
_NP = 10240
_RPT = _NP // _NS
_CHK = 64
_NG = 4
_GCH = 40
_NBF = 4
_EPAD = _NW * _NG * _GCH * _CHK

_BLK = 1024
_NBLK = _NP // _BLK

_SQRT_HALF = 0.7071067811865476


def _sc_mesh():
    return plsc.VectorSubcoreMesh(
        core_axis_name="c", subcore_axis_name="s",
        num_cores=_NC, num_subcores=_NS)


def _sc_degree(dstw, ones_hbm, zrows):
    @functools.partial(
        pl.kernel,
        out_type=jax.ShapeDtypeStruct((_NC, _NP, _D), jnp.float32),
        mesh=_sc_mesh(),
        scratch_types=[
            pltpu.VMEM((_NG, _GCH, _CHK), jnp.int32),
            pltpu.VMEM((_CHK, _D), jnp.float32),
            pltpu.VMEM_SHARED((_NP, _D), jnp.float32),
            pltpu.SemaphoreType.DMA,
        ],
    )
    def deg_kernel(dstw_ref, ones_ref, zeros_ref, degp_ref, dst_v, ones_v, hist_sh,
                   sem):
        c = lax.axis_index("c")
        s = lax.axis_index("s")
        wid = c * _NS + s
        pltpu.sync_copy(zeros_ref, hist_sh.at[pl.ds(s * _RPT, _RPT)])
        pltpu.sync_copy(dstw_ref.at[wid], dst_v)
        pltpu.sync_copy(ones_ref, ones_v)
        plsc.subcore_barrier()

        cps = [pltpu.async_copy(ones_v, hist_sh.at[dst_v.at[g, k]], sem, add=True)
               for g in range(_NG) for k in range(_GCH)]
        for cp in cps:
            cp.wait()

        plsc.subcore_barrier()
        pltpu.sync_copy(hist_sh.at[pl.ds(s * _RPT, _RPT)],
                        degp_ref.at[c, pl.ds(s * _RPT, _RPT)])

    return deg_kernel(dstw, ones_hbm, zrows)


def _sc_scatter(srcw, dstw, y, zrows):
    @functools.partial(
        pl.kernel,
        out_type=jax.ShapeDtypeStruct((_NC, _NP, _D), jnp.float32),
        mesh=_sc_mesh(),
        scratch_types=[
            pltpu.VMEM((_GCH, _CHK), jnp.int32),
            pltpu.VMEM((_GCH, _CHK), jnp.int32),
            pltpu.VMEM_SHARED((_NP, _D), jnp.float32),
        ] + [pltpu.VMEM((_CHK, _D), jnp.float32)] * _NBF
          + [pltpu.SemaphoreType.DMA] * (2 * _NBF + 1),
    )
    def scat_kernel(srcw_ref, dstw_ref, y_ref, zr_ref, out_ref,
                    srcg, dstg, acc_sh, *rest):
        bufs = rest[:_NBF]
        sems = rest[_NBF:2 * _NBF]
        ssems = rest[2 * _NBF:3 * _NBF]
        sem_z = rest[3 * _NBF]
        c = lax.axis_index("c")
        s = lax.axis_index("s")
        wid = c * _NS + s
        zcp = pltpu.async_copy(zr_ref, acc_sh.at[pl.ds(s * _RPT, _RPT)], sem_z)

        for g in range(_NG):
            pltpu.sync_copy(srcw_ref.at[wid, g], srcg)
            pltpu.sync_copy(dstw_ref.at[wid, g], dstg)
            cps = {}
            scps = {}

            def fire(j):
                cps[j] = pltpu.async_copy(
                    y_ref.at[srcg.at[j]], bufs[j % _NBF], sems[j % _NBF])

            for j in range(_NBF):
                fire(j)
            if g == 0:
                zcp.wait()
                plsc.subcore_barrier()
            for k in range(_GCH):
                if k >= 1 and k - 1 + _NBF < _GCH:
                    scps[k - 1].wait()
                    fire(k - 1 + _NBF)
                cps[k].wait()
                scps[k] = pltpu.async_copy(
                    bufs[k % _NBF], acc_sh.at[dstg.at[k]],
                    ssems[k % _NBF], add=True)
            for k in range(_GCH - _NBF, _GCH):
                scps[k].wait()

        plsc.subcore_barrier()
        pltpu.sync_copy(acc_sh.at[pl.ds(s * _RPT, _RPT)],
                        out_ref.at[c, pl.ds(s * _RPT, _RPT)])

    return scat_kernel(srcw, dstw, y, zrows)


def _dis_block(degp_blk):
    deg = degp_blk[0] + degp_blk[1] + 1.0
    return lax.rsqrt(deg[:, 0:1])


def _tc_prep(x, W1, degp):
    def body(x_ref, w_ref, degp_ref, y_ref, dis_ref):
        dis = _dis_block(degp_ref)
        dis_ref[...] = dis
        y_ref[...] = jnp.dot(x_ref[...], w_ref[...],
                             preferred_element_type=jnp.float32) * dis

    return pl.pallas_call(
        body,
        grid=(_NBLK,),
        in_specs=[
            pl.BlockSpec((_BLK, _D), lambda i: (i, 0)),
            pl.BlockSpec((_D, _D), lambda i: (0, 0)),
            pl.BlockSpec((_NC, _BLK, _D), lambda i: (0, i, 0)),
        ],
        out_specs=[
            pl.BlockSpec((_BLK, _D), lambda i: (i, 0)),
            pl.BlockSpec((_BLK, 1), lambda i: (i, 0)),
        ],
        out_shape=[
            jax.ShapeDtypeStruct((_NP, _D), jnp.float32),
            jax.ShapeDtypeStruct((_NP, 1), jnp.float32),
        ],
    )(x, W1, degp)


def _tc_assemble(Sp, y, dis_v, b):
    def body(sp_ref, y_ref, dis_ref, b_ref, t_ref, st_ref):
        i = pl.program_id(0)
        dis = dis_ref[...]
        t = (sp_ref[0] + sp_ref[1] + y_ref[...]) * dis + b_ref[...]
        t_ref[...] = t

        rid = lax.broadcasted_iota(jnp.int32, (_BLK, 1), 0) + i * _BLK
        tm = jnp.where(rid < _N, t, 0.0)

        @pl.when(i == 0)
        def _():
            st_ref[...] = jnp.zeros_like(st_ref)

        st_ref[0:1, :] += jnp.sum(tm, axis=0, keepdims=True)
        st_ref[1:2, :] += jnp.sum(tm * tm, axis=0, keepdims=True)

    return pl.pallas_call(
        body,
        grid=(_NBLK,),
        in_specs=[
            pl.BlockSpec((_NC, _BLK, _D), lambda i: (0, i, 0)),
            pl.BlockSpec((_BLK, _D), lambda i: (i, 0)),
            pl.BlockSpec((_BLK, 1), lambda i: (i, 0)),
            pl.BlockSpec((1, _D), lambda i: (0, 0)),
        ],
        out_specs=[
            pl.BlockSpec((_BLK, _D), lambda i: (i, 0)),
            pl.BlockSpec((8, _D), lambda i: (0, 0)),
        ],
        out_shape=[
            jax.ShapeDtypeStruct((_NP, _D), jnp.float32),
            jax.ShapeDtypeStruct((8, _D), jnp.float32),
        ],
    )(Sp, y, dis_v, b)


def _tc_bn_gelu_mm(t, st, g, be, W, dis_v):
    def body(t_ref, st_ref, g_ref, be_ref, w_ref, dis_ref, y_ref):
        dis = dis_ref[...]
        mean = st_ref[0:1, :] * (1.0 / _N)
        var = st_ref[1:2, :] * (1.0 / _N) - mean * mean
        h = g_ref[...] * (t_ref[...] - mean) * lax.rsqrt(var + _EPS) + be_ref[...]
        h = 0.5 * h * (1.0 + lax.erf(h * _SQRT_HALF))
        y_ref[...] = jnp.dot(h, w_ref[...],
                             preferred_element_type=jnp.float32) * dis

    return pl.pallas_call(
        body,
        grid=(_NBLK,),
        in_specs=[
            pl.BlockSpec((_BLK, _D), lambda i: (i, 0)),
            pl.BlockSpec((8, _D), lambda i: (0, 0)),
            pl.BlockSpec((1, _D), lambda i: (0, 0)),
            pl.BlockSpec((1, _D), lambda i: (0, 0)),
            pl.BlockSpec((_D, _D), lambda i: (0, 0)),
            pl.BlockSpec((_BLK, 1), lambda i: (i, 0)),
        ],
        out_specs=pl.BlockSpec((_BLK, _D), lambda i: (i, 0)),
        out_shape=jax.ShapeDtypeStruct((_NP, _D), jnp.float32),
    )(t, st, g, be, W, dis_v)


def _tc_final(Sp, y, dis_v, b):
    def body(sp_ref, y_ref, dis_ref, b_ref, o_ref):
        dis = dis_ref[...]
        o_ref[...] = (sp_ref[0] + sp_ref[1] + y_ref[...]) * dis + b_ref[...]

    return pl.pallas_call(
        body,
        grid=(_NBLK,),
        in_specs=[
            pl.BlockSpec((_NC, _BLK, _D), lambda i: (0, i, 0)),
            pl.BlockSpec((_BLK, _D), lambda i: (i, 0)),
            pl.BlockSpec((_BLK, 1), lambda i: (i, 0)),
            pl.BlockSpec((1, _D), lambda i: (0, 0)),
        ],
        out_specs=pl.BlockSpec((_BLK, _D), lambda i: (i, 0)),
        out_shape=jax.ShapeDtypeStruct((_N, _D), jnp.float32),
    )(Sp, y, dis_v, b)


def kernel(x, edge_index, W1, b1, g1, be1, W2, b2, g2, be2, W3, b3):
    npad = _EPAD - _E
    pad_idx = _N + (jnp.arange(npad, dtype=jnp.int32) % (_NP - _N))
    src = jnp.concatenate([edge_index[0], pad_idx]).reshape(_NW, _NG, _GCH, _CHK)
    dst = jnp.concatenate([edge_index[1], pad_idx]).reshape(_NW, _NG, _GCH, _CHK)
    ones_hbm = jnp.ones((_CHK, _D), jnp.float32)
    zrows = jnp.zeros((_RPT, _D), jnp.float32)

    degp = _sc_degree(dst, ones_hbm, zrows)

    y1, dis_v = _tc_prep(x, W1, degp)
    S1 = _sc_scatter(src, dst, y1, zrows)
    t1, st1 = _tc_assemble(S1, y1, dis_v, b1.reshape(1, _D))
    y2 = _tc_bn_gelu_mm(t1, st1, g1.reshape(1, _D), be1.reshape(1, _D), W2, dis_v)

    S2 = _sc_scatter(src, dst, y2, zrows)
    t2, st2 = _tc_assemble(S2, y2, dis_v, b2.reshape(1, _D))
    y3 = _tc_bn_gelu_mm(t2, st2, g2.reshape(1, _D), be2.reshape(1, _D), W3, dis_v)

    S3 = _sc_scatter(src, dst, y3, zrows)
    return _tc_final(S3, y3, dis_v, b3.reshape(1, _D))
